# Initial kernel scaffold; baseline (speedup 1.0000x reference)
import jax
import jax.numpy as jnp
from jax.experimental import pallas as pl


def kernel(x, target):
    raise NotImplementedError("write your pallas kernel here")


# speedup vs baseline: 2.4605x; 2.4605x over previous
import math

import jax
import jax.numpy as jnp
import numpy as np
from jax import lax
from jax.experimental import pallas as pl
from jax.experimental.pallas import tpu as pltpu
from jax.experimental.pallas import tpu_sc as plsc

_SIZE = 32000
_N = 2048
_PAD = 0
_S32 = float(np.float32(0.1 / (_SIZE - 2)))
_C_ROW = (_SIZE - 2) * _S32 * math.log(_S32) + 0.9 * math.log(0.9)
_COEF = 0.9 - _S32

_NC, _NS, _L = 2, 16, 16
_NW = _NC * _NS
_RPW = _N // _NW


def _sc_body(xflat, tgt, out, tgt_v, idx_v, val_v, acc_v, sem):
    wid = lax.axis_index("s") * _NC + lax.axis_index("c")
    base = wid * _RPW
    pltpu.sync_copy(tgt.at[pl.ds(base, _RPW)], tgt_v)
    for j in range(_RPW // _L):
        t16 = tgt_v[pl.ds(j * _L, _L)]
        rows = lax.iota(jnp.int32, _L) + (base + j * _L)
        idx_v[pl.ds(j * _L, _L)] = rows * _SIZE + t16
    pltpu.async_copy(xflat.at[idx_v], val_v, sem).wait()
    acc = jnp.zeros((_L,), jnp.float32)
    for j in range(_RPW // _L):
        t16 = tgt_v[pl.ds(j * _L, _L)]
        v16 = val_v[pl.ds(j * _L, _L)]
        acc = acc + jnp.where(
            t16 != _PAD,
            jnp.float32(_C_ROW) - jnp.float32(_COEF) * v16,
            jnp.float32(0.0),
        )
    acc_v[...] = acc
    pltpu.sync_copy(acc_v, out.at[pl.ds(wid * _L, _L)])


_sc_gather = pl.kernel(
    _sc_body,
    out_type=jax.ShapeDtypeStruct((_NW * _L,), jnp.float32),
    mesh=plsc.VectorSubcoreMesh(core_axis_name="c", subcore_axis_name="s"),
    scratch_types=[
        pltpu.VMEM((_RPW,), jnp.int32),
        pltpu.VMEM((_RPW,), jnp.int32),
        pltpu.VMEM((_RPW,), jnp.float32),
        pltpu.VMEM((_L,), jnp.float32),
        pltpu.SemaphoreType.DMA,
    ],
)

_RB = 256
_CB = 6400
_NI = _N // _RB
_NJ = _SIZE // _CB


def _tc_body(tgt_ref, x_ref, scp_ref, out_ref, acc_ref):
    i = pl.program_id(0)
    j = pl.program_id(1)

    @pl.when((i == 0) & (j == 0))
    def _init():
        acc_ref[0] = 0.0

    mask = (tgt_ref[...] != _PAD).astype(jnp.float32)
    rowsum = jnp.sum(x_ref[...], axis=1, keepdims=True)
    part = jnp.sum(rowsum * mask)
    col0 = jnp.sum(x_ref[:, 0:1] * mask)
    acc_ref[0] = acc_ref[0] + (part - jnp.where(j == 0, col0, 0.0))

    @pl.when((i == _NI - 1) & (j == _NJ - 1))
    def _emit():
        loss = jnp.sum(scp_ref[...]) - jnp.float32(_S32) * acc_ref[0]
        out_ref[...] = jnp.reshape(loss, (1, 1))


_tc_reduce = pl.pallas_call(
    _tc_body,
    grid=(_NI, _NJ),
    in_specs=[
        pl.BlockSpec((_RB, 1), lambda i, j: (i, 0)),
        pl.BlockSpec((_RB, _CB), lambda i, j: (i, j)),
        pl.BlockSpec((_NW, _L), lambda i, j: (0, 0)),
    ],
    out_specs=pl.BlockSpec((1, 1), lambda i, j: (0, 0)),
    out_shape=jax.ShapeDtypeStruct((1, 1), jnp.float32),
    scratch_shapes=[pltpu.SMEM((1,), jnp.float32)],
)


def kernel(x, target):
    tgt32 = target.astype(jnp.int32)
    scp = _sc_gather(jnp.reshape(x, (_N * _SIZE,)), tgt32)
    out = _tc_reduce(jnp.reshape(tgt32, (_N, 1)), x, jnp.reshape(scp, (_NW, _L)))
    return out[0, 0]


# speedup vs baseline: 2.4613x; 1.0003x over previous
import math

import jax
import jax.numpy as jnp
import numpy as np
from jax import lax
from jax.experimental import pallas as pl
from jax.experimental.pallas import tpu as pltpu
from jax.experimental.pallas import tpu_sc as plsc

_SIZE = 32000
_N = 2048
_PAD = 0
_S32 = float(np.float32(0.1 / (_SIZE - 2)))
_C_ROW = (_SIZE - 2) * _S32 * math.log(_S32) + 0.9 * math.log(0.9)
_COEF = 0.9 - _S32

_NC, _NS, _L = 2, 16, 16
_NW = _NC * _NS
_RPW = _N // _NW


def _sc_body(xflat, tgt, out, tgt_v, idx_v, val_v, acc_v, sem):
    wid = lax.axis_index("s") * _NC + lax.axis_index("c")
    base = wid * _RPW
    pltpu.sync_copy(tgt.at[pl.ds(base, _RPW)], tgt_v)
    for j in range(_RPW // _L):
        t16 = tgt_v[pl.ds(j * _L, _L)]
        rows = lax.iota(jnp.int32, _L) + (base + j * _L)
        idx_v[pl.ds(j * _L, _L)] = rows * _SIZE + t16
    pltpu.async_copy(xflat.at[idx_v], val_v, sem).wait()
    acc = jnp.zeros((_L,), jnp.float32)
    for j in range(_RPW // _L):
        t16 = tgt_v[pl.ds(j * _L, _L)]
        v16 = val_v[pl.ds(j * _L, _L)]
        acc = acc + jnp.where(
            t16 != _PAD,
            jnp.float32(_C_ROW) - jnp.float32(_COEF) * v16,
            jnp.float32(0.0),
        )
    acc_v[...] = acc
    pltpu.sync_copy(acc_v, out.at[pl.ds(wid * _L, _L)])


_sc_gather = pl.kernel(
    _sc_body,
    out_type=jax.ShapeDtypeStruct((_NW * _L,), jnp.float32),
    mesh=plsc.VectorSubcoreMesh(core_axis_name="c", subcore_axis_name="s"),
    scratch_types=[
        pltpu.VMEM((_RPW,), jnp.int32),
        pltpu.VMEM((_RPW,), jnp.int32),
        pltpu.VMEM((_RPW,), jnp.float32),
        pltpu.VMEM((_L,), jnp.float32),
        pltpu.SemaphoreType.DMA,
    ],
)

_RB = 128
_NI = _N // _RB


def _tc_body(tgt_ref, x_ref, scp_ref, out_ref, acc_ref):
    i = pl.program_id(0)

    @pl.when(i == 0)
    def _init():
        acc_ref[0] = 0.0

    mask = (tgt_ref[...] != _PAD).astype(jnp.float32)
    rowsum = jnp.sum(x_ref[...], axis=1, keepdims=True)
    acc_ref[0] = acc_ref[0] + jnp.sum((rowsum - x_ref[:, 0:1]) * mask)

    @pl.when(i == _NI - 1)
    def _emit():
        loss = jnp.sum(scp_ref[...]) - jnp.float32(_S32) * acc_ref[0]
        out_ref[...] = jnp.reshape(loss, (1, 1))


_tc_reduce = pl.pallas_call(
    _tc_body,
    grid=(_NI,),
    in_specs=[
        pl.BlockSpec((_RB, 1), lambda i: (i, 0)),
        pl.BlockSpec((_RB, _SIZE), lambda i: (i, 0)),
        pl.BlockSpec((_NW, _L), lambda i: (0, 0)),
    ],
    out_specs=pl.BlockSpec((1, 1), lambda i: (0, 0)),
    out_shape=jax.ShapeDtypeStruct((1, 1), jnp.float32),
    scratch_shapes=[pltpu.SMEM((1,), jnp.float32)],
)


def kernel(x, target):
    tgt32 = target.astype(jnp.int32)
    scp = _sc_gather(jnp.reshape(x, (_N * _SIZE,)), tgt32)
    out = _tc_reduce(jnp.reshape(tgt32, (_N, 1)), x, jnp.reshape(scp, (_NW, _L)))
    return out[0, 0]


# speedup vs baseline: 2.4696x; 1.0034x over previous
import math

import jax
import jax.numpy as jnp
import numpy as np
from jax import lax
from jax.experimental import pallas as pl
from jax.experimental.pallas import tpu as pltpu
from jax.experimental.pallas import tpu_sc as plsc

_SIZE = 32000
_N = 2048
_PAD = 0
_S32 = float(np.float32(0.1 / (_SIZE - 2)))
_C_ROW = (_SIZE - 2) * _S32 * math.log(_S32) + 0.9 * math.log(0.9)
_COEF = 0.9 - _S32

_NC, _NS, _L = 2, 16, 16
_NW = _NC * _NS
_RPW = _N // _NW


def _sc_body(xflat, tgt, out, tgt_v, idx_v, val_v, acc_v, sem):
    wid = lax.axis_index("s") * _NC + lax.axis_index("c")
    base = wid * _RPW
    pltpu.sync_copy(tgt.at[pl.ds(base, _RPW)], tgt_v)
    for j in range(_RPW // _L):
        t16 = tgt_v[pl.ds(j * _L, _L)]
        rows = lax.iota(jnp.int32, _L) + (base + j * _L)
        idx_v[pl.ds(j * _L, _L)] = rows * _SIZE + t16
    pltpu.async_copy(xflat.at[idx_v], val_v, sem).wait()
    acc = jnp.zeros((_L,), jnp.float32)
    for j in range(_RPW // _L):
        t16 = tgt_v[pl.ds(j * _L, _L)]
        v16 = val_v[pl.ds(j * _L, _L)]
        acc = acc + jnp.where(
            t16 != _PAD,
            jnp.float32(_C_ROW) - jnp.float32(_COEF) * v16,
            jnp.float32(0.0),
        )
    acc_v[...] = acc
    pltpu.sync_copy(acc_v, out.at[pl.ds(wid * _L, _L)])


_sc_gather = pl.kernel(
    _sc_body,
    out_type=jax.ShapeDtypeStruct((_NW * _L,), jnp.float32),
    mesh=plsc.VectorSubcoreMesh(core_axis_name="c", subcore_axis_name="s"),
    scratch_types=[
        pltpu.VMEM((_RPW,), jnp.int32),
        pltpu.VMEM((_RPW,), jnp.int32),
        pltpu.VMEM((_RPW,), jnp.float32),
        pltpu.VMEM((_L,), jnp.float32),
        pltpu.SemaphoreType.DMA,
    ],
)

_ST = 2
_RB = 64
_NPS = _N // _ST
_NI = _NPS // _RB


def _tc_body(tgt_ref, x0_ref, x1_ref, scp_ref, out_ref, acc_ref):
    i = pl.program_id(0)

    @pl.when(i == 0)
    def _init():
        acc_ref[0] = 0.0

    part = jnp.float32(0.0)
    for k, x_ref in enumerate((x0_ref, x1_ref)):
        mask = (tgt_ref[0, :, k : k + 1] != _PAD).astype(jnp.float32)
        rowsum = jnp.sum(x_ref[0], axis=1, keepdims=True)
        part = part + jnp.sum((rowsum - x_ref[0, :, 0:1]) * mask)
    acc_ref[0] = acc_ref[0] + part

    @pl.when(i == _NI - 1)
    def _emit():
        loss = jnp.sum(scp_ref[...]) - jnp.float32(_S32) * acc_ref[0]
        out_ref[...] = jnp.reshape(loss, (1, 1))


_tc_reduce = pl.pallas_call(
    _tc_body,
    grid=(_NI,),
    in_specs=[
        pl.BlockSpec((1, _RB, _ST), lambda i: (0, i, 0)),
        pl.BlockSpec((1, _RB, _SIZE), lambda i: (0, i, 0)),
        pl.BlockSpec((1, _RB, _SIZE), lambda i: (1, i, 0)),
        pl.BlockSpec((_NW, _L), lambda i: (0, 0)),
    ],
    out_specs=pl.BlockSpec((1, 1), lambda i: (0, 0)),
    out_shape=jax.ShapeDtypeStruct((1, 1), jnp.float32),
    scratch_shapes=[pltpu.SMEM((1,), jnp.float32)],
)


def kernel(x, target):
    tgt32 = target.astype(jnp.int32)
    scp = _sc_gather(jnp.reshape(x, (_N * _SIZE,)), tgt32)
    xs = jnp.reshape(x, (_ST, _NPS, _SIZE))
    tgts = jnp.reshape(tgt32, (_ST, _NPS)).T[None]
    out = _tc_reduce(tgts, xs, xs, jnp.reshape(scp, (_NW, _L)))
    return out[0, 0]
